# baseline (speedup 1.0000x reference)
import functools

import jax
import jax.numpy as jnp
from jax import lax
from jax.experimental import pallas as pl
from jax.experimental.pallas import tpu as pltpu
from jax.experimental.pallas import tpu_sc as plsc

N = 10000
D = 128
H = 128
C = 16
E = 320000

NC = 2
NS = 16
NW = NC * NS
EPT = 10240
EPAD = EPT * NW
CK = 128
NCH = EPT // CK
CK2 = 64
NCH2 = EPT // CK2
NPAD = 10112
RPC = NPAD // NS

_mesh = plsc.VectorSubcoreMesh(core_axis_name="c", subcore_axis_name="s",
                               num_cores=NC, num_subcores=NS)
_params = pltpu.CompilerParams(needs_layout_passes=False)


def _zero_rows(zsrc, acc_sh, base):
    nz = zsrc.shape[0]
    nfull, rem = RPC // nz, RPC % nz
    for k in range(nfull):
        pltpu.sync_copy(zsrc, acc_sh.at[pl.ds(base + k * nz, nz)])
    if rem:
        pltpu.sync_copy(zsrc.at[pl.ds(0, rem)], acc_sh.at[pl.ds(base + nfull * nz, rem)])


def _write_rows(acc_sh, out_hbm, cid, base, bounce):
    nz = bounce.shape[0]
    nfull, rem = RPC // nz, RPC % nz
    for k in range(nfull):
        pltpu.sync_copy(acc_sh.at[pl.ds(base + k * nz, nz)], bounce)
        pltpu.sync_copy(bounce, out_hbm.at[cid, pl.ds(base + k * nz, nz)])
    if rem:
        pltpu.sync_copy(acc_sh.at[pl.ds(base + nfull * nz, rem)],
                        bounce.at[pl.ds(0, rem)])
        pltpu.sync_copy(bounce.at[pl.ds(0, rem)],
                        out_hbm.at[cid, pl.ds(base + nfull * nz, rem)])


def _zero_1d(ref):
    z16 = jnp.zeros((16,), jnp.float32)

    def zbody(i, carry):
        ref[pl.ds(i * 16, 16)] = z16
        return carry

    lax.fori_loop(0, ref.shape[0] // 16, zbody, 0)


_DEG_SIG = dict(
    out_type=jax.ShapeDtypeStruct((NW, NPAD), jnp.float32),
    mesh=_mesh,
    compiler_params=_params,
    scratch_types=[
        pltpu.VMEM((EPT,), jnp.int32),
        pltpu.VMEM((NPAD,), jnp.float32),
    ],
)


def _deg_body(dst_hbm, degp_hbm, dstv, degl):
    cid = lax.axis_index("c")
    sid = lax.axis_index("s")
    wid = cid * NS + sid
    pltpu.sync_copy(dst_hbm.at[wid], dstv)
    _zero_1d(degl)
    one16 = jnp.ones((16,), jnp.float32)

    def body(g, carry):
        d16 = dstv[pl.ds(g * 16, 16)]
        plsc.addupdate_scatter(degl, [d16], one16)
        return carry

    lax.fori_loop(0, EPT // 16, body, 0)
    pltpu.sync_copy(degl, degp_hbm.at[wid])


_AGG_SIG = dict(
    out_type=jax.ShapeDtypeStruct((NC, NPAD, H), jnp.float32),
    mesh=_mesh,
    compiler_params=_params,
    scratch_types=[
        pltpu.VMEM((16, CK), jnp.int32),
        pltpu.VMEM((16, CK), jnp.int32),
        pltpu.VMEM((CK, H), jnp.float32),
        pltpu.VMEM((CK, H), jnp.float32),
        pltpu.VMEM_SHARED((NPAD, H), jnp.float32),
        pltpu.SemaphoreType.DMA,
        pltpu.SemaphoreType.DMA,
        pltpu.SemaphoreType.DMA,
        pltpu.SemaphoreType.DMA,
    ],
)


def _agg_body(src_hbm, dst_hbm, h_hbm, z128_hbm, sp_hbm,
              srcv, dstv, rows0, rows1, s_sh, semg0, semg1, sems0, sems1):
    cid = lax.axis_index("c")
    sid = lax.axis_index("s")
    wid = cid * NS + sid
    base = sid * RPC
    pltpu.sync_copy(z128_hbm, rows0)
    _zero_rows(rows0, s_sh, base)
    plsc.subcore_barrier()

    def body(jj, carry):
        j0 = jj * 2
        j1 = j0 + 1
        c0 = pltpu.async_copy(h_hbm.at[srcv.at[j0]], rows0, semg0)
        c1 = pltpu.async_copy(h_hbm.at[srcv.at[j1]], rows1, semg1)
        c0.wait()
        s0 = pltpu.async_copy(rows0, s_sh.at[dstv.at[j0]], sems0, add=True)
        c1.wait()
        s1 = pltpu.async_copy(rows1, s_sh.at[dstv.at[j1]], sems1, add=True)
        s0.wait()
        s1.wait()
        return carry

    for q in range(NCH // 16):
        pltpu.sync_copy(src_hbm.at[wid, pl.ds(q * 16, 16)], srcv)
        pltpu.sync_copy(dst_hbm.at[wid, pl.ds(q * 16, 16)], dstv)
        lax.fori_loop(0, 8, body, 0)
    plsc.subcore_barrier()
    _write_rows(s_sh, sp_hbm, cid, base, rows0)


_AGG_U_SIG = dict(
    out_type=(jax.ShapeDtypeStruct((NC, NPAD, H), jnp.float32),
              jax.ShapeDtypeStruct((NW, NPAD), jnp.float32)),
    mesh=_mesh,
    compiler_params=_params,
    scratch_types=[
        pltpu.VMEM((16, CK2), jnp.int32),
        pltpu.VMEM((16, CK2), jnp.int32),
        pltpu.VMEM((CK2, H), jnp.float32),
        pltpu.VMEM((CK2, H), jnp.float32),
        pltpu.VMEM((NPAD,), jnp.float32),
        pltpu.VMEM((NPAD,), jnp.float32),
        pltpu.VMEM_SHARED((NPAD, H), jnp.float32),
        pltpu.SemaphoreType.DMA,
        pltpu.SemaphoreType.DMA,
        pltpu.SemaphoreType.DMA,
        pltpu.SemaphoreType.DMA,
    ],
)


def _agg_u_body(src_hbm, dst_hbm, h_hbm, dis1_hbm, z128_hbm,
                sp_hbm, up_hbm,
                srcv, dstv, rows0, rows1, disv, ul, s_sh,
                semg0, semg1, sems0, sems1):
    cid = lax.axis_index("c")
    sid = lax.axis_index("s")
    wid = cid * NS + sid
    base = sid * RPC
    pltpu.sync_copy(dis1_hbm, disv)
    pltpu.sync_copy(z128_hbm.at[pl.ds(0, CK2)], rows0)
    _zero_rows(rows0, s_sh, base)
    _zero_1d(ul)
    plsc.subcore_barrier()

    def body(jj, carry):
        j0 = jj * 2
        j1 = j0 + 1
        c0 = pltpu.async_copy(h_hbm.at[srcv.at[j0]], rows0, semg0)
        c1 = pltpu.async_copy(h_hbm.at[srcv.at[j1]], rows1, semg1)
        for j in (j0, j1):
            for g in range(CK2 // 16):
                dst16 = dstv[j, pl.ds(g * 16, 16)]
                src16 = srcv[j, pl.ds(g * 16, 16)]
                vals = plsc.load_gather(disv, [dst16])
                plsc.addupdate_scatter(ul, [src16], vals)
        c0.wait()
        s0 = pltpu.async_copy(rows0, s_sh.at[dstv.at[j0]], sems0, add=True)
        c1.wait()
        s1 = pltpu.async_copy(rows1, s_sh.at[dstv.at[j1]], sems1, add=True)
        s0.wait()
        s1.wait()
        return carry

    for q in range(NCH2 // 16):
        pltpu.sync_copy(src_hbm.at[wid, pl.ds(q * 16, 16)], srcv)
        pltpu.sync_copy(dst_hbm.at[wid, pl.ds(q * 16, 16)], dstv)
        lax.fori_loop(0, 8, body, 0)
    plsc.subcore_barrier()
    pltpu.sync_copy(ul, up_hbm.at[wid])
    _write_rows(s_sh, sp_hbm, cid, base, rows0)


_agg_g_kernel = None


def _agg_g_body(src_hbm, dst_hbm, h_hbm, z128_hbm, sp_hbm,
                srcv, dstv, rows0, rows1, s_sh, semg0, semg1, sems0, sems1):
    cid = lax.axis_index("c")
    sid = lax.axis_index("s")
    wid = cid * NS + sid
    base = sid * RPC

    def body(jj, carry):
        j0 = jj * 2
        j1 = j0 + 1
        c0 = pltpu.async_copy(h_hbm.at[srcv.at[j0]], rows0, semg0)
        c1 = pltpu.async_copy(h_hbm.at[srcv.at[j1]], rows1, semg1)
        c0.wait()
        c1.wait()
        return carry

    for q in range(NCH // 16):
        pltpu.sync_copy(src_hbm.at[wid, pl.ds(q * 16, 16)], srcv)
        pltpu.sync_copy(dst_hbm.at[wid, pl.ds(q * 16, 16)], dstv)
        lax.fori_loop(0, 8, body, 0)
    _write_rows(s_sh, sp_hbm, cid, base, rows0)


_agg_s_kernel = None


def _agg_s_body(src_hbm, dst_hbm, h_hbm, z128_hbm, sp_hbm,
                srcv, dstv, rows0, rows1, s_sh, semg0, semg1, sems0, sems1):
    cid = lax.axis_index("c")
    sid = lax.axis_index("s")
    wid = cid * NS + sid
    base = sid * RPC
    pltpu.sync_copy(z128_hbm, rows0)
    pltpu.sync_copy(z128_hbm, rows1)
    _zero_rows(rows0, s_sh, base)
    plsc.subcore_barrier()

    def body(jj, carry):
        j0 = jj * 2
        j1 = j0 + 1
        s0 = pltpu.async_copy(rows0, s_sh.at[dstv.at[j0]], sems0, add=True)
        s1 = pltpu.async_copy(rows1, s_sh.at[dstv.at[j1]], sems1, add=True)
        s0.wait()
        s1.wait()
        return carry

    for q in range(NCH // 16):
        pltpu.sync_copy(src_hbm.at[wid, pl.ds(q * 16, 16)], srcv)
        pltpu.sync_copy(dst_hbm.at[wid, pl.ds(q * 16, 16)], dstv)
        lax.fori_loop(0, 8, body, 0)
    plsc.subcore_barrier()
    _write_rows(s_sh, sp_hbm, cid, base, rows0)

_deg_kernel = pl.kernel(_deg_body, **_DEG_SIG)
_agg_g_kernel = pl.kernel(_agg_g_body, **_AGG_SIG)
_agg_s_kernel = pl.kernel(_agg_s_body, **_AGG_SIG)
_agg_kernel = pl.kernel(_agg_body, **_AGG_SIG)
_agg_u_kernel = pl.kernel(_agg_u_body, **_AGG_U_SIG)


def _tc1_body(x_ref, w1_ref, degp_ref, h_ref, dis16_ref, dis1_ref):
    deg = jnp.sum(degp_ref[...], axis=0)[:, None] + 1.0
    rid = lax.broadcasted_iota(jnp.int32, (NPAD, 1), 0)
    dis = lax.rsqrt(deg) * (rid < N).astype(jnp.float32)
    z = jnp.dot(x_ref[...], w1_ref[...], preferred_element_type=jnp.float32)
    h_ref[...] = z * dis
    dis16_ref[...] = jnp.broadcast_to(dis, (NPAD, 16))
    dis1_ref[...] = dis[:, 0]


def _tc2_body(sp_ref, h1_ref, dis16_ref, b1_ref, w2_ref, out_ref):
    dis = dis16_ref[:, 0:1]
    s = sp_ref[0] + sp_ref[1]
    ha = jnp.maximum(dis * (s + h1_ref[...]) + b1_ref[...], 0.0)
    out_ref[...] = dis * jnp.dot(ha, w2_ref[...],
                                 preferred_element_type=jnp.float32)


def _tc3_body(sp_ref, h2_ref, dis16_ref, up_ref, b2_ref, w3_ref, b3_ref,
              wl_ref, bl_ref, out_ref):
    dis = dis16_ref[:, 0:1]
    s = sp_ref[0] + sp_ref[1]
    hb = jnp.maximum(dis * (s + h2_ref[...]) + b2_ref[...], 0.0)
    u = jnp.sum(up_ref[...], axis=0)[:, None]
    w = dis * (u + dis)
    v = jnp.sum(hb * w, axis=0, keepdims=True) * (1.0 / N)
    g = jnp.dot(v, w3_ref[...], preferred_element_type=jnp.float32) + b3_ref[...]
    out_ref[...] = jnp.dot(g, wl_ref[...],
                           preferred_element_type=jnp.float32) + bl_ref[...]


_tc1 = pl.pallas_call(
    _tc1_body,
    out_shape=(jax.ShapeDtypeStruct((NPAD, H), jnp.float32),
               jax.ShapeDtypeStruct((NPAD, 16), jnp.float32),
               jax.ShapeDtypeStruct((NPAD,), jnp.float32)))

_tc2 = pl.pallas_call(
    _tc2_body,
    out_shape=jax.ShapeDtypeStruct((NPAD, H), jnp.float32))

_tc3 = pl.pallas_call(
    _tc3_body,
    out_shape=jax.ShapeDtypeStruct((1, C), jnp.float32))


def kernel(x, edge_index, W1, b1, W2, b2, W3, b3, Wl, bl):
    src = edge_index[0].astype(jnp.int32)
    dst = edge_index[1].astype(jnp.int32)
    src_p = jnp.concatenate([src, jnp.zeros((EPAD - E,), jnp.int32)])
    dst_p = jnp.concatenate([dst, jnp.full((EPAD - E,), N, jnp.int32)])
    src3 = src_p.reshape(NW, NCH, CK)
    dst3 = dst_p.reshape(NW, NCH, CK)
    z128 = jnp.zeros((CK, H), jnp.float32)
    xp = jnp.concatenate([x, jnp.zeros((NPAD - N, D), x.dtype)])

    degp = _deg_kernel(dst_p.reshape(NW, EPT))
    h1t, dis16, dis1 = _tc1(xp, W1, degp)
    s1p = _agg_kernel(src3, dst3, h1t, z128)
    sg = _agg_g_kernel(src3, dst3, s1p[0], z128)
    ss = _agg_s_kernel(src3, dst3, sg[0], z128)
    s1b = _agg_kernel(src3, dst3, ss[0], z128)
    h1t = h1t + 0.0 * s1b[0]
    h2t = _tc2(s1p, h1t, dis16, b1.reshape(1, H), W2)
    s2p, up = _agg_u_kernel(src_p.reshape(NW, NCH2, CK2),
                            dst_p.reshape(NW, NCH2, CK2), h2t, dis1, z128)
    out = _tc3(s2p, h2t, dis16, up, b2.reshape(1, H), W3,
               b3.reshape(1, H), Wl, bl.reshape(1, C))
    return out


# speedup vs baseline: 1.8795x; 1.8795x over previous
import functools

import jax
import jax.numpy as jnp
from jax import lax
from jax.experimental import pallas as pl
from jax.experimental.pallas import tpu as pltpu
from jax.experimental.pallas import tpu_sc as plsc

N = 10000
D = 128
H = 128
C = 16
E = 320000

NC = 2
NS = 16
NW = NC * NS
EPT = 10240
EPAD = EPT * NW
CK = 128
NCH = EPT // CK
CK2 = 64
NCH2 = EPT // CK2
NPAD = 10112
RPC = NPAD // NS

_mesh = plsc.VectorSubcoreMesh(core_axis_name="c", subcore_axis_name="s",
                               num_cores=NC, num_subcores=NS)
_params = pltpu.CompilerParams(needs_layout_passes=False)


def _zero_rows(zsrc, acc_sh, base):
    nz = zsrc.shape[0]
    nfull, rem = RPC // nz, RPC % nz
    for k in range(nfull):
        pltpu.sync_copy(zsrc, acc_sh.at[pl.ds(base + k * nz, nz)])
    if rem:
        pltpu.sync_copy(zsrc.at[pl.ds(0, rem)], acc_sh.at[pl.ds(base + nfull * nz, rem)])


def _write_rows(acc_sh, out_hbm, cid, base, bounce):
    nz = bounce.shape[0]
    nfull, rem = RPC // nz, RPC % nz
    for k in range(nfull):
        pltpu.sync_copy(acc_sh.at[pl.ds(base + k * nz, nz)], bounce)
        pltpu.sync_copy(bounce, out_hbm.at[cid, pl.ds(base + k * nz, nz)])
    if rem:
        pltpu.sync_copy(acc_sh.at[pl.ds(base + nfull * nz, rem)],
                        bounce.at[pl.ds(0, rem)])
        pltpu.sync_copy(bounce.at[pl.ds(0, rem)],
                        out_hbm.at[cid, pl.ds(base + nfull * nz, rem)])


def _zero_1d(ref):
    z16 = jnp.zeros((16,), jnp.float32)

    def zbody(i, carry):
        ref[pl.ds(i * 16, 16)] = z16
        return carry

    lax.fori_loop(0, ref.shape[0] // 16, zbody, 0)


_DEG_SIG = dict(
    out_type=jax.ShapeDtypeStruct((NW, NPAD), jnp.float32),
    mesh=_mesh,
    compiler_params=_params,
    scratch_types=[
        pltpu.VMEM((EPT,), jnp.int32),
        pltpu.VMEM((NPAD,), jnp.float32),
    ],
)


def _deg_body(dst_hbm, degp_hbm, dstv, degl):
    cid = lax.axis_index("c")
    sid = lax.axis_index("s")
    wid = cid * NS + sid
    pltpu.sync_copy(dst_hbm.at[wid], dstv)
    _zero_1d(degl)
    one16 = jnp.ones((16,), jnp.float32)

    def body(g, carry):
        d16 = dstv[pl.ds(g * 16, 16)]
        plsc.addupdate_scatter(degl, [d16], one16)
        return carry

    lax.fori_loop(0, EPT // 16, body, 0)
    pltpu.sync_copy(degl, degp_hbm.at[wid])


_AGG_SIG = dict(
    out_type=jax.ShapeDtypeStruct((NC, NPAD, H), jnp.float32),
    mesh=_mesh,
    compiler_params=_params,
    scratch_types=[
        pltpu.VMEM((16, CK), jnp.int32),
        pltpu.VMEM((16, CK), jnp.int32),
        pltpu.VMEM((CK, H), jnp.float32),
        pltpu.VMEM((CK, H), jnp.float32),
        pltpu.VMEM_SHARED((NPAD, H), jnp.float32),
        pltpu.SemaphoreType.DMA,
        pltpu.SemaphoreType.DMA,
        pltpu.SemaphoreType.DMA,
        pltpu.SemaphoreType.DMA,
    ],
)


def _agg_body(src_hbm, dst_hbm, h_hbm, z128_hbm, sp_hbm,
              srcv, dstv, rows0, rows1, s_sh, semg0, semg1, sems0, sems1):
    cid = lax.axis_index("c")
    sid = lax.axis_index("s")
    wid = cid * NS + sid
    base = sid * RPC
    tbl = h_hbm.at[cid]
    pltpu.sync_copy(z128_hbm, rows0)
    _zero_rows(rows0, s_sh, base)
    plsc.subcore_barrier()

    def body(jj, carry):
        j0 = jj * 2
        j1 = j0 + 1
        c0 = pltpu.async_copy(tbl.at[srcv.at[j0]], rows0, semg0)
        c1 = pltpu.async_copy(tbl.at[srcv.at[j1]], rows1, semg1)
        c0.wait()
        s0 = pltpu.async_copy(rows0, s_sh.at[dstv.at[j0]], sems0, add=True)
        c1.wait()
        s1 = pltpu.async_copy(rows1, s_sh.at[dstv.at[j1]], sems1, add=True)
        s0.wait()
        s1.wait()
        return carry

    for q in range(NCH // 16):
        pltpu.sync_copy(src_hbm.at[wid, pl.ds(q * 16, 16)], srcv)
        pltpu.sync_copy(dst_hbm.at[wid, pl.ds(q * 16, 16)], dstv)
        lax.fori_loop(0, 8, body, 0)
    plsc.subcore_barrier()
    _write_rows(s_sh, sp_hbm, cid, base, rows0)


_AGG_U_SIG = dict(
    out_type=(jax.ShapeDtypeStruct((NC, NPAD, H), jnp.float32),
              jax.ShapeDtypeStruct((NW, NPAD), jnp.float32)),
    mesh=_mesh,
    compiler_params=_params,
    scratch_types=[
        pltpu.VMEM((16, CK2), jnp.int32),
        pltpu.VMEM((16, CK2), jnp.int32),
        pltpu.VMEM((CK2, H), jnp.float32),
        pltpu.VMEM((CK2, H), jnp.float32),
        pltpu.VMEM((NPAD,), jnp.float32),
        pltpu.VMEM((NPAD,), jnp.float32),
        pltpu.VMEM_SHARED((NPAD, H), jnp.float32),
        pltpu.SemaphoreType.DMA,
        pltpu.SemaphoreType.DMA,
        pltpu.SemaphoreType.DMA,
        pltpu.SemaphoreType.DMA,
    ],
)


def _agg_u_body(src_hbm, dst_hbm, h_hbm, dis1_hbm, z128_hbm,
                sp_hbm, up_hbm,
                srcv, dstv, rows0, rows1, disv, ul, s_sh,
                semg0, semg1, sems0, sems1):
    cid = lax.axis_index("c")
    sid = lax.axis_index("s")
    wid = cid * NS + sid
    base = sid * RPC
    tbl = h_hbm.at[cid]
    pltpu.sync_copy(dis1_hbm, disv)
    pltpu.sync_copy(z128_hbm.at[pl.ds(0, CK2)], rows0)
    _zero_rows(rows0, s_sh, base)
    _zero_1d(ul)
    plsc.subcore_barrier()

    def body(jj, carry):
        j0 = jj * 2
        j1 = j0 + 1
        c0 = pltpu.async_copy(tbl.at[srcv.at[j0]], rows0, semg0)
        c1 = pltpu.async_copy(tbl.at[srcv.at[j1]], rows1, semg1)
        for j in (j0, j1):
            for g in range(CK2 // 16):
                dst16 = dstv[j, pl.ds(g * 16, 16)]
                src16 = srcv[j, pl.ds(g * 16, 16)]
                vals = plsc.load_gather(disv, [dst16])
                plsc.addupdate_scatter(ul, [src16], vals)
        c0.wait()
        s0 = pltpu.async_copy(rows0, s_sh.at[dstv.at[j0]], sems0, add=True)
        c1.wait()
        s1 = pltpu.async_copy(rows1, s_sh.at[dstv.at[j1]], sems1, add=True)
        s0.wait()
        s1.wait()
        return carry

    for q in range(NCH2 // 16):
        pltpu.sync_copy(src_hbm.at[wid, pl.ds(q * 16, 16)], srcv)
        pltpu.sync_copy(dst_hbm.at[wid, pl.ds(q * 16, 16)], dstv)
        lax.fori_loop(0, 8, body, 0)
    plsc.subcore_barrier()
    pltpu.sync_copy(ul, up_hbm.at[wid])
    _write_rows(s_sh, sp_hbm, cid, base, rows0)


_deg_kernel = pl.kernel(_deg_body, **_DEG_SIG)
_agg_kernel = pl.kernel(_agg_body, **_AGG_SIG)
_agg_u_kernel = pl.kernel(_agg_u_body, **_AGG_U_SIG)


def _tc1_body(x_ref, w1_ref, degp_ref, h_ref, dis16_ref, dis1_ref):
    deg = jnp.sum(degp_ref[...], axis=0)[:, None] + 1.0
    rid = lax.broadcasted_iota(jnp.int32, (NPAD, 1), 0)
    dis = lax.rsqrt(deg) * (rid < N).astype(jnp.float32)
    z = jnp.dot(x_ref[...], w1_ref[...], preferred_element_type=jnp.float32)
    ht = z * dis
    h_ref[0] = ht
    h_ref[1] = ht
    dis16_ref[...] = jnp.broadcast_to(dis, (NPAD, 16))
    dis1_ref[...] = dis[:, 0]


def _tc2_body(sp_ref, h1_ref, dis16_ref, b1_ref, w2_ref, out_ref):
    dis = dis16_ref[:, 0:1]
    s = sp_ref[0] + sp_ref[1]
    ha = jnp.maximum(dis * (s + h1_ref[0]) + b1_ref[...], 0.0)
    ht = dis * jnp.dot(ha, w2_ref[...], preferred_element_type=jnp.float32)
    out_ref[0] = ht
    out_ref[1] = ht


def _tc3_body(sp_ref, h2_ref, dis16_ref, up_ref, b2_ref, w3_ref, b3_ref,
              wl_ref, bl_ref, out_ref):
    dis = dis16_ref[:, 0:1]
    s = sp_ref[0] + sp_ref[1]
    hb = jnp.maximum(dis * (s + h2_ref[0]) + b2_ref[...], 0.0)
    u = jnp.sum(up_ref[...], axis=0)[:, None]
    w = dis * (u + dis)
    v = jnp.sum(hb * w, axis=0, keepdims=True) * (1.0 / N)
    g = jnp.dot(v, w3_ref[...], preferred_element_type=jnp.float32) + b3_ref[...]
    out_ref[...] = jnp.dot(g, wl_ref[...],
                           preferred_element_type=jnp.float32) + bl_ref[...]


_tc1 = pl.pallas_call(
    _tc1_body,
    out_shape=(jax.ShapeDtypeStruct((2, NPAD, H), jnp.float32),
               jax.ShapeDtypeStruct((NPAD, 16), jnp.float32),
               jax.ShapeDtypeStruct((NPAD,), jnp.float32)))

_tc2 = pl.pallas_call(
    _tc2_body,
    out_shape=jax.ShapeDtypeStruct((2, NPAD, H), jnp.float32))

_tc3 = pl.pallas_call(
    _tc3_body,
    out_shape=jax.ShapeDtypeStruct((1, C), jnp.float32))


def kernel(x, edge_index, W1, b1, W2, b2, W3, b3, Wl, bl):
    src = edge_index[0].astype(jnp.int32)
    dst = edge_index[1].astype(jnp.int32)
    src_p = jnp.concatenate([src, jnp.zeros((EPAD - E,), jnp.int32)])
    dst_p = jnp.concatenate([dst, jnp.full((EPAD - E,), N, jnp.int32)])
    src3 = src_p.reshape(NW, NCH, CK)
    dst3 = dst_p.reshape(NW, NCH, CK)
    z128 = jnp.zeros((CK, H), jnp.float32)
    xp = jnp.concatenate([x, jnp.zeros((NPAD - N, D), x.dtype)])

    degp = _deg_kernel(dst_p.reshape(NW, EPT))
    h1t, dis16, dis1 = _tc1(xp, W1, degp)
    s1p = _agg_kernel(src3, dst3, h1t, z128)
    h2t = _tc2(s1p, h1t, dis16, b1.reshape(1, H), W2)
    s2p, up = _agg_u_kernel(src_p.reshape(NW, NCH2, CK2),
                            dst_p.reshape(NW, NCH2, CK2), h2t, dis1, z128)
    out = _tc3(s2p, h2t, dis16, up, b2.reshape(1, H), W3,
               b3.reshape(1, H), Wl, bl.reshape(1, C))
    return out


# speedup vs baseline: 2.0894x; 1.1117x over previous
import functools

import jax
import jax.numpy as jnp
from jax import lax
from jax.experimental import pallas as pl
from jax.experimental.pallas import tpu as pltpu
from jax.experimental.pallas import tpu_sc as plsc

N = 10000
D = 128
H = 128
C = 16
E = 320000

NC = 2
NS = 16
NW = NC * NS
EPT = 10240
EPAD = EPT * NW
CK = 128
NCH = EPT // CK
CK2 = 64
NCH2 = EPT // CK2
NPAD = 10112
RPC = NPAD // NS

_mesh = plsc.VectorSubcoreMesh(core_axis_name="c", subcore_axis_name="s",
                               num_cores=NC, num_subcores=NS)
_params = pltpu.CompilerParams(needs_layout_passes=False)


def _zero_rows(zsrc, acc_sh, base):
    nz = zsrc.shape[0]
    nfull, rem = RPC // nz, RPC % nz
    for k in range(nfull):
        pltpu.sync_copy(zsrc, acc_sh.at[pl.ds(base + k * nz, nz)])
    if rem:
        pltpu.sync_copy(zsrc.at[pl.ds(0, rem)], acc_sh.at[pl.ds(base + nfull * nz, rem)])


def _write_rows(acc_sh, out_hbm, cid, base, bounce):
    nz = bounce.shape[0]
    nfull, rem = RPC // nz, RPC % nz
    for k in range(nfull):
        pltpu.sync_copy(acc_sh.at[pl.ds(base + k * nz, nz)], bounce)
        pltpu.sync_copy(bounce, out_hbm.at[cid, pl.ds(base + k * nz, nz)])
    if rem:
        pltpu.sync_copy(acc_sh.at[pl.ds(base + nfull * nz, rem)],
                        bounce.at[pl.ds(0, rem)])
        pltpu.sync_copy(bounce.at[pl.ds(0, rem)],
                        out_hbm.at[cid, pl.ds(base + nfull * nz, rem)])


def _zero_1d(ref):
    z16 = jnp.zeros((16,), jnp.float32)

    def zbody(i, carry):
        ref[pl.ds(i * 16, 16)] = z16
        return carry

    lax.fori_loop(0, ref.shape[0] // 16, zbody, 0)


_DEG_SIG = dict(
    out_type=jax.ShapeDtypeStruct((NW, NPAD), jnp.float32),
    mesh=_mesh,
    compiler_params=_params,
    scratch_types=[
        pltpu.VMEM((EPT,), jnp.int32),
        pltpu.VMEM((NPAD,), jnp.float32),
    ],
)


def _deg_body(dst_hbm, degp_hbm, dstv, degl):
    cid = lax.axis_index("c")
    sid = lax.axis_index("s")
    wid = cid * NS + sid
    pltpu.sync_copy(dst_hbm.at[wid], dstv)
    _zero_1d(degl)
    one16 = jnp.ones((16,), jnp.float32)

    def body(g, carry):
        d16 = dstv[pl.ds(g * 16, 16)]
        plsc.addupdate_scatter(degl, [d16], one16)
        return carry

    lax.fori_loop(0, EPT // 16, body, 0)
    pltpu.sync_copy(degl, degp_hbm.at[wid])


NBUF = 4

_AGG_SIG = dict(
    out_type=jax.ShapeDtypeStruct((NC, NPAD, H), jnp.float32),
    mesh=_mesh,
    compiler_params=_params,
    scratch_types=(
        [pltpu.VMEM((16, CK2), jnp.int32),
         pltpu.VMEM((16, CK2), jnp.int32)]
        + [pltpu.VMEM((CK2, H), jnp.float32)] * NBUF
        + [pltpu.VMEM_SHARED((NPAD, H), jnp.float32)]
        + [pltpu.SemaphoreType.DMA] * (2 * NBUF)
    ),
)


def _agg_body(src_hbm, dst_hbm, h_hbm, z128_hbm, sp_hbm,
              srcv, dstv, *rest):
    rows = rest[:NBUF]
    s_sh = rest[NBUF]
    semg = rest[NBUF + 1:NBUF + 1 + NBUF]
    sems = rest[NBUF + 1 + NBUF:]
    cid = lax.axis_index("c")
    sid = lax.axis_index("s")
    wid = cid * NS + sid
    base = sid * RPC
    pltpu.sync_copy(z128_hbm.at[pl.ds(0, CK2)], rows[0])
    _zero_rows(rows[0], s_sh, base)
    plsc.subcore_barrier()

    def body(jj, carry):
        cs = []
        for b in range(NBUF):
            j = jj * NBUF + b
            cs.append(pltpu.async_copy(h_hbm.at[srcv.at[j]], rows[b], semg[b]))
        ss = []
        for b in range(NBUF):
            j = jj * NBUF + b
            cs[b].wait()
            ss.append(pltpu.async_copy(rows[b], s_sh.at[dstv.at[j]],
                                       sems[b], add=True))
        for b in range(NBUF):
            ss[b].wait()
        return carry

    for q in range(NCH2 // 16):
        pltpu.sync_copy(src_hbm.at[wid, pl.ds(q * 16, 16)], srcv)
        pltpu.sync_copy(dst_hbm.at[wid, pl.ds(q * 16, 16)], dstv)
        lax.fori_loop(0, 16 // NBUF, body, 0)
    plsc.subcore_barrier()
    _write_rows(s_sh, sp_hbm, cid, base, rows[0])


_AGG_U_SIG = dict(
    out_type=(jax.ShapeDtypeStruct((NC, NPAD, H), jnp.float32),
              jax.ShapeDtypeStruct((NW, NPAD), jnp.float32)),
    mesh=_mesh,
    compiler_params=_params,
    scratch_types=[
        pltpu.VMEM((16, CK2), jnp.int32),
        pltpu.VMEM((16, CK2), jnp.int32),
        pltpu.VMEM((CK2, H), jnp.float32),
        pltpu.VMEM((CK2, H), jnp.float32),
        pltpu.VMEM((NPAD,), jnp.float32),
        pltpu.VMEM((NPAD,), jnp.float32),
        pltpu.VMEM_SHARED((NPAD, H), jnp.float32),
        pltpu.SemaphoreType.DMA,
        pltpu.SemaphoreType.DMA,
        pltpu.SemaphoreType.DMA,
        pltpu.SemaphoreType.DMA,
    ],
)


def _agg_u_body(src_hbm, dst_hbm, h_hbm, dis1_hbm, z128_hbm,
                sp_hbm, up_hbm,
                srcv, dstv, rows0, rows1, disv, ul, s_sh,
                semg0, semg1, sems0, sems1):
    cid = lax.axis_index("c")
    sid = lax.axis_index("s")
    wid = cid * NS + sid
    base = sid * RPC
    pltpu.sync_copy(dis1_hbm, disv)
    pltpu.sync_copy(z128_hbm.at[pl.ds(0, CK2)], rows0)
    _zero_rows(rows0, s_sh, base)
    _zero_1d(ul)
    plsc.subcore_barrier()

    def body(jj, carry):
        j0 = jj * 2
        j1 = j0 + 1
        c0 = pltpu.async_copy(h_hbm.at[srcv.at[j0]], rows0, semg0)
        c1 = pltpu.async_copy(h_hbm.at[srcv.at[j1]], rows1, semg1)
        for j in (j0, j1):
            for g in range(CK2 // 16):
                dst16 = dstv[j, pl.ds(g * 16, 16)]
                src16 = srcv[j, pl.ds(g * 16, 16)]
                vals = plsc.load_gather(disv, [dst16])
                plsc.addupdate_scatter(ul, [src16], vals)
        c0.wait()
        s0 = pltpu.async_copy(rows0, s_sh.at[dstv.at[j0]], sems0, add=True)
        c1.wait()
        s1 = pltpu.async_copy(rows1, s_sh.at[dstv.at[j1]], sems1, add=True)
        s0.wait()
        s1.wait()
        return carry

    for q in range(NCH2 // 16):
        pltpu.sync_copy(src_hbm.at[wid, pl.ds(q * 16, 16)], srcv)
        pltpu.sync_copy(dst_hbm.at[wid, pl.ds(q * 16, 16)], dstv)
        lax.fori_loop(0, 8, body, 0)
    plsc.subcore_barrier()
    pltpu.sync_copy(ul, up_hbm.at[wid])
    _write_rows(s_sh, sp_hbm, cid, base, rows0)


_deg_kernel = pl.kernel(_deg_body, **_DEG_SIG)
_agg_kernel = pl.kernel(_agg_body, **_AGG_SIG)
_agg_u_kernel = pl.kernel(_agg_u_body, **_AGG_U_SIG)


def _tc1_body(x_ref, w1_ref, degp_ref, h_ref, dis16_ref, dis1_ref):
    deg = jnp.sum(degp_ref[...], axis=0)[:, None] + 1.0
    rid = lax.broadcasted_iota(jnp.int32, (NPAD, 1), 0)
    dis = lax.rsqrt(deg) * (rid < N).astype(jnp.float32)
    z = jnp.dot(x_ref[...], w1_ref[...], preferred_element_type=jnp.float32)
    h_ref[...] = z * dis
    dis16_ref[...] = jnp.broadcast_to(dis, (NPAD, 16))
    dis1_ref[...] = dis[:, 0]


def _tc2_body(sp_ref, h1_ref, dis16_ref, b1_ref, w2_ref, out_ref):
    dis = dis16_ref[:, 0:1]
    s = sp_ref[0] + sp_ref[1]
    ha = jnp.maximum(dis * (s + h1_ref[...]) + b1_ref[...], 0.0)
    out_ref[...] = dis * jnp.dot(ha, w2_ref[...],
                                 preferred_element_type=jnp.float32)


def _tc3_body(sp_ref, h2_ref, dis16_ref, up_ref, b2_ref, w3_ref, b3_ref,
              wl_ref, bl_ref, out_ref):
    dis = dis16_ref[:, 0:1]
    s = sp_ref[0] + sp_ref[1]
    hb = jnp.maximum(dis * (s + h2_ref[...]) + b2_ref[...], 0.0)
    u = jnp.sum(up_ref[...], axis=0)[:, None]
    w = dis * (u + dis)
    v = jnp.sum(hb * w, axis=0, keepdims=True) * (1.0 / N)
    g = jnp.dot(v, w3_ref[...], preferred_element_type=jnp.float32) + b3_ref[...]
    out_ref[...] = jnp.dot(g, wl_ref[...],
                           preferred_element_type=jnp.float32) + bl_ref[...]


_tc1 = pl.pallas_call(
    _tc1_body,
    out_shape=(jax.ShapeDtypeStruct((NPAD, H), jnp.float32),
               jax.ShapeDtypeStruct((NPAD, 16), jnp.float32),
               jax.ShapeDtypeStruct((NPAD,), jnp.float32)))

_tc2 = pl.pallas_call(
    _tc2_body,
    out_shape=jax.ShapeDtypeStruct((NPAD, H), jnp.float32))

_tc3 = pl.pallas_call(
    _tc3_body,
    out_shape=jax.ShapeDtypeStruct((1, C), jnp.float32))


def kernel(x, edge_index, W1, b1, W2, b2, W3, b3, Wl, bl):
    src = edge_index[0].astype(jnp.int32)
    dst = edge_index[1].astype(jnp.int32)
    src_p = jnp.concatenate([src, jnp.zeros((EPAD - E,), jnp.int32)])
    dst_p = jnp.concatenate([dst, jnp.full((EPAD - E,), N, jnp.int32)])
    src3 = src_p.reshape(NW, NCH, CK)
    dst3 = dst_p.reshape(NW, NCH, CK)
    z128 = jnp.zeros((CK, H), jnp.float32)
    xp = jnp.concatenate([x, jnp.zeros((NPAD - N, D), x.dtype)])

    degp = _deg_kernel(dst_p.reshape(NW, EPT))
    h1t, dis16, dis1 = _tc1(xp, W1, degp)
    s1p = _agg_kernel(src_p.reshape(NW, NCH2, CK2),
                      dst_p.reshape(NW, NCH2, CK2), h1t, z128)
    h2t = _tc2(s1p, h1t, dis16, b1.reshape(1, H), W2)
    s2p, up = _agg_u_kernel(src_p.reshape(NW, NCH2, CK2),
                            dst_p.reshape(NW, NCH2, CK2), h2t, dis1, z128)
    out = _tc3(s2p, h2t, dis16, up, b2.reshape(1, H), W3,
               b3.reshape(1, H), Wl, bl.reshape(1, C))
    return out
